# baseline (speedup 1.0000x reference)
import functools

import jax
import jax.numpy as jnp
from jax import lax
from jax.experimental import pallas as pl
from jax.experimental.pallas import tpu as pltpu, tpu_sc as plsc

NC = 2
NS = 16
NW = NC * NS
L = 16
CB = 128

F32 = jnp.float32
I32 = jnp.int32


def _sc_mesh():
    return plsc.VectorSubcoreMesh(
        core_axis_name="c", subcore_axis_name="s", num_cores=NC, num_subcores=NS
    )


def _zero_vmem_rows(ref, nrows, ncols):
    z = jnp.zeros((L,), F32)

    def body(i, _):
        for g in range(ncols // L):
            ref[i, pl.ds(g * L, L)] = z
        return 0

    lax.fori_loop(0, nrows, body, 0)


def _sc_degree(dst3, n_pad, chunks):
    tn = n_pad // NS
    KF = 8

    @functools.partial(
        pl.kernel,
        out_type=jax.ShapeDtypeStruct((NC, n_pad), F32),
        mesh=_sc_mesh(),
        scratch_types=[
            pltpu.VMEM((chunks, CB), I32),
            pltpu.VMEM((CB,), F32),
            pltpu.VMEM((tn,), F32),
            pltpu.SemaphoreType.DMA,
            pltpu.VMEM_SHARED((n_pad,), F32),
        ],
    )
    def k(dst_hbm, deg_hbm, idx_v, ones_v, slice_v, sem, deg_sh):
        c = lax.axis_index("c")
        s = lax.axis_index("s")
        w = c * NS + s
        cp_idx = pltpu.async_copy(dst_hbm.at[w], idx_v, sem)
        one = jnp.ones((L,), F32)
        zero = jnp.zeros((L,), F32)
        for g in range(CB // L):
            ones_v[pl.ds(g * L, L)] = one

        def zbody(i, _):
            slice_v[pl.ds(i * L, L)] = zero
            return 0

        lax.fori_loop(0, tn // L, zbody, 0)
        pltpu.sync_copy(slice_v, deg_sh.at[pl.ds(s * tn, tn)])
        cp_idx.wait()
        plsc.subcore_barrier()

        @pl.loop(0, chunks, step=KF)
        def ebody(g):
            for j in range(KF):
                pltpu.sync_copy(ones_v, deg_sh.at[idx_v.at[g + j]], add=True)

        plsc.subcore_barrier()

        pltpu.sync_copy(deg_sh.at[pl.ds(s * tn, tn)], slice_v)
        pltpu.sync_copy(slice_v, deg_hbm.at[c, pl.ds(s * tn, tn)])

    return k(dst3)


def _sc_aggregate(xwn, src3, dst3, n_pad, chunks):
    tn = n_pad // NS
    D = xwn.shape[1]
    NB = 2
    NH = 2
    hc = chunks // NH

    @functools.partial(
        pl.kernel,
        out_type=jax.ShapeDtypeStruct((NC, n_pad, D), F32),
        mesh=_sc_mesh(),
        scratch_types=[
            pltpu.VMEM((hc, CB), I32),
            pltpu.VMEM((hc, CB), I32),
            [pltpu.VMEM((CB, D), F32) for _ in range(NB)],
            [pltpu.SemaphoreType.DMA for _ in range(NB)],
            pltpu.SemaphoreType.DMA,
            pltpu.VMEM_SHARED((n_pad, D), F32),
        ],
    )
    def k(xwn_hbm, src_hbm, dst_hbm, s_hbm,
          src_i, dst_i, rows, gsem, isem, agg_sh):
        c = lax.axis_index("c")
        s = lax.axis_index("s")
        w = c * NS + s

        def load_idx(h):
            pltpu.async_copy(src_hbm.at[w, pl.ds(h * hc, hc)], src_i, isem)
            pltpu.async_copy(dst_hbm.at[w, pl.ds(h * hc, hc)], dst_i, isem)

        def drain_idx():
            pltpu.make_async_copy(src_hbm.at[w, pl.ds(0, hc)], src_i,
                                  isem).wait()
            pltpu.make_async_copy(dst_hbm.at[w, pl.ds(0, hc)], dst_i,
                                  isem).wait()

        load_idx(0)
        _zero_vmem_rows(rows[0], CB, D)
        for kk in range(tn // CB):
            pltpu.sync_copy(rows[0], agg_sh.at[pl.ds(s * tn + kk * CB, CB)])
        drain_idx()
        plsc.subcore_barrier()

        def gather(g, b):
            return pltpu.async_copy(xwn_hbm.at[src_i.at[g]], rows[b], gsem[b])

        def wait_gather(g, b):
            pltpu.make_async_copy(xwn_hbm.at[src_i.at[g]], rows[b],
                                  gsem[b]).wait()

        def scatter(g, b):
            pltpu.sync_copy(rows[b], agg_sh.at[dst_i.at[g]], add=True)

        for h in range(NH):
            for b in range(NB):
                gather(b, b)

            def wave(i, _):
                g0 = i * NB
                for b in range(NB):
                    wait_gather(g0 + b, b)
                    scatter(g0 + b, b)
                    gather(g0 + NB + b, b)
                return 0

            lax.fori_loop(0, hc // NB - 1, wave, 0)
            g0 = hc - NB
            for b in range(NB):
                wait_gather(g0 + b, b)
                scatter(g0 + b, b)
            if h + 1 < NH:
                load_idx(h + 1)
                drain_idx()
        plsc.subcore_barrier()

        for kk in range(tn // CB):
            sl = pl.ds(s * tn + kk * CB, CB)
            pltpu.sync_copy(agg_sh.at[sl], rows[0])
            pltpu.sync_copy(rows[0], s_hbm.at[c, sl])

    return k(xwn, src3, dst3)


def _tc_prescale(x_p, W, degs3):
    n_pad, D = x_p.shape
    BR = 1024

    def body(x_ref, w_ref, d0_ref, d1_ref, xwn_ref, dinv_ref):
        deg = d0_ref[0] + d1_ref[0] + 1.0
        dinv = lax.rsqrt(deg)
        xw = jnp.dot(x_ref[...], w_ref[...], preferred_element_type=F32)
        xwn_ref[...] = xw * dinv
        dinv_ref[...] = dinv

    return pl.pallas_call(
        body,
        grid=(n_pad // BR,),
        in_specs=[
            pl.BlockSpec((BR, D), lambda i: (i, 0)),
            pl.BlockSpec((D, D), lambda i: (0, 0)),
            pl.BlockSpec((1, BR, 1), lambda i: (0, i, 0)),
            pl.BlockSpec((1, BR, 1), lambda i: (1, i, 0)),
        ],
        out_specs=[
            pl.BlockSpec((BR, D), lambda i: (i, 0)),
            pl.BlockSpec((BR, 1), lambda i: (i, 0)),
        ],
        out_shape=[
            jax.ShapeDtypeStruct((n_pad, D), F32),
            jax.ShapeDtypeStruct((n_pad, 1), F32),
        ],
    )(x_p, W, degs3, degs3)


def _tc_combine(s_all, xwn, dinv, b, n):
    D = xwn.shape[1]
    BR = 1000

    def body(s0_ref, s1_ref, xwn_ref, dinv_ref, b_ref, out_ref):
        acc = s0_ref[0] + s1_ref[0] + xwn_ref[...]
        out_ref[...] = jnp.maximum(acc * dinv_ref[...] + b_ref[...], 0.0)

    return pl.pallas_call(
        body,
        grid=(n // BR,),
        in_specs=[
            pl.BlockSpec((1, BR, D), lambda i: (0, i, 0)),
            pl.BlockSpec((1, BR, D), lambda i: (1, i, 0)),
            pl.BlockSpec((BR, D), lambda i: (i, 0)),
            pl.BlockSpec((BR, 1), lambda i: (i, 0)),
            pl.BlockSpec((D,), lambda i: (0,)),
        ],
        out_specs=pl.BlockSpec((BR, D), lambda i: (i, 0)),
        out_shape=jax.ShapeDtypeStruct((n, D), F32),
    )(s_all, s_all, xwn, dinv, b)


def kernel(x, edge_index, W, b):
    n, D = x.shape
    E = edge_index.shape[1]
    n_pad = ((n + 255) // 256) * 256
    chunks = ((-(-E // (NW * CB)) + 7) // 8) * 8
    e_pad = NW * CB * chunks

    pad = jnp.full((e_pad - E,), n, I32)
    src3 = jnp.concatenate([edge_index[0], pad]).reshape(NW, chunks, CB)
    dst3 = jnp.concatenate([edge_index[1], pad]).reshape(NW, chunks, CB)
    x_p = jnp.pad(x, ((0, n_pad - n), (0, 0)))

    degs = _sc_degree(dst3, n_pad, chunks)
    xwn, dinv = _tc_prescale(x_p, W, degs.reshape(NC, n_pad, 1))
    s_all = _sc_aggregate(xwn, src3, dst3, n_pad, chunks)
    return _tc_combine(s_all, xwn, dinv, b, n)


# speedup vs baseline: 1.0309x; 1.0309x over previous
import functools

import jax
import jax.numpy as jnp
from jax import lax
from jax.experimental import pallas as pl
from jax.experimental.pallas import tpu as pltpu, tpu_sc as plsc

NC = 2
NS = 16
NW = NC * NS
L = 16
CB = 64

F32 = jnp.float32
I32 = jnp.int32


def _sc_mesh():
    return plsc.VectorSubcoreMesh(
        core_axis_name="c", subcore_axis_name="s", num_cores=NC, num_subcores=NS
    )


def _zero_vmem_rows(ref, nrows, ncols):
    z = jnp.zeros((L,), F32)

    def body(i, _):
        for g in range(ncols // L):
            ref[i, pl.ds(g * L, L)] = z
        return 0

    lax.fori_loop(0, nrows, body, 0)


def _sc_degree(dst3, n_pad, chunks):
    tn = n_pad // NS
    KF = 8

    @functools.partial(
        pl.kernel,
        out_type=jax.ShapeDtypeStruct((NC, n_pad), F32),
        mesh=_sc_mesh(),
        scratch_types=[
            pltpu.VMEM((chunks, CB), I32),
            pltpu.VMEM((CB,), F32),
            pltpu.VMEM((tn,), F32),
            pltpu.SemaphoreType.DMA,
            pltpu.VMEM_SHARED((n_pad,), F32),
        ],
    )
    def k(dst_hbm, deg_hbm, idx_v, ones_v, slice_v, sem, deg_sh):
        c = lax.axis_index("c")
        s = lax.axis_index("s")
        w = c * NS + s
        cp_idx = pltpu.async_copy(dst_hbm.at[w], idx_v, sem)
        one = jnp.ones((L,), F32)
        zero = jnp.zeros((L,), F32)
        for g in range(CB // L):
            ones_v[pl.ds(g * L, L)] = one

        def zbody(i, _):
            slice_v[pl.ds(i * L, L)] = zero
            return 0

        lax.fori_loop(0, tn // L, zbody, 0)
        pltpu.sync_copy(slice_v, deg_sh.at[pl.ds(s * tn, tn)])
        cp_idx.wait()
        plsc.subcore_barrier()

        @pl.loop(0, chunks, step=KF)
        def ebody(g):
            for j in range(KF):
                pltpu.sync_copy(ones_v, deg_sh.at[idx_v.at[g + j]], add=True)

        plsc.subcore_barrier()

        pltpu.sync_copy(deg_sh.at[pl.ds(s * tn, tn)], slice_v)
        pltpu.sync_copy(slice_v, deg_hbm.at[c, pl.ds(s * tn, tn)])

    return k(dst3)


def _sc_aggregate(xwn, src3, dst3, n_pad, chunks):
    tn = n_pad // NS
    D = xwn.shape[1]
    NB = 4
    NH = 4
    hc = chunks // NH

    @functools.partial(
        pl.kernel,
        out_type=jax.ShapeDtypeStruct((NC, n_pad, D), F32),
        mesh=_sc_mesh(),
        scratch_types=[
            pltpu.VMEM((hc, CB), I32),
            pltpu.VMEM((hc, CB), I32),
            [pltpu.VMEM((CB, D), F32) for _ in range(NB)],
            [pltpu.SemaphoreType.DMA for _ in range(NB)],
            pltpu.SemaphoreType.DMA,
            pltpu.VMEM_SHARED((n_pad, D), F32),
        ],
    )
    def k(xwn_hbm, src_hbm, dst_hbm, s_hbm,
          src_i, dst_i, rows, gsem, isem, agg_sh):
        c = lax.axis_index("c")
        s = lax.axis_index("s")
        w = c * NS + s

        def load_idx(h):
            pltpu.async_copy(src_hbm.at[w, pl.ds(h * hc, hc)], src_i, isem)
            pltpu.async_copy(dst_hbm.at[w, pl.ds(h * hc, hc)], dst_i, isem)

        def drain_idx():
            pltpu.make_async_copy(src_hbm.at[w, pl.ds(0, hc)], src_i,
                                  isem).wait()
            pltpu.make_async_copy(dst_hbm.at[w, pl.ds(0, hc)], dst_i,
                                  isem).wait()

        load_idx(0)
        _zero_vmem_rows(rows[0], CB, D)
        for kk in range(tn // CB):
            pltpu.sync_copy(rows[0], agg_sh.at[pl.ds(s * tn + kk * CB, CB)])
        drain_idx()
        plsc.subcore_barrier()

        def gather(g, b):
            return pltpu.async_copy(xwn_hbm.at[src_i.at[g]], rows[b], gsem[b])

        def wait_gather(g, b):
            pltpu.make_async_copy(xwn_hbm.at[src_i.at[g]], rows[b],
                                  gsem[b]).wait()

        def scatter(g, b):
            pltpu.sync_copy(rows[b], agg_sh.at[dst_i.at[g]], add=True)

        for h in range(NH):
            for b in range(NB):
                gather(b, b)

            def wave(i, _):
                g0 = i * NB
                for b in range(NB):
                    wait_gather(g0 + b, b)
                    scatter(g0 + b, b)
                    gather(g0 + NB + b, b)
                return 0

            lax.fori_loop(0, hc // NB - 1, wave, 0)
            g0 = hc - NB
            for b in range(NB):
                wait_gather(g0 + b, b)
                scatter(g0 + b, b)
            if h + 1 < NH:
                load_idx(h + 1)
                drain_idx()
        plsc.subcore_barrier()

        for kk in range(tn // CB):
            sl = pl.ds(s * tn + kk * CB, CB)
            pltpu.sync_copy(agg_sh.at[sl], rows[0])
            pltpu.sync_copy(rows[0], s_hbm.at[c, sl])

    return k(xwn, src3, dst3)


def _tc_prescale(x_p, W, degs3):
    n_pad, D = x_p.shape
    BR = 1024

    def body(x_ref, w_ref, d0_ref, d1_ref, xwn_ref, dinv_ref):
        deg = d0_ref[0] + d1_ref[0] + 1.0
        dinv = lax.rsqrt(deg)
        xw = jnp.dot(x_ref[...], w_ref[...], preferred_element_type=F32)
        xwn_ref[...] = xw * dinv
        dinv_ref[...] = dinv

    return pl.pallas_call(
        body,
        grid=(n_pad // BR,),
        in_specs=[
            pl.BlockSpec((BR, D), lambda i: (i, 0)),
            pl.BlockSpec((D, D), lambda i: (0, 0)),
            pl.BlockSpec((1, BR, 1), lambda i: (0, i, 0)),
            pl.BlockSpec((1, BR, 1), lambda i: (1, i, 0)),
        ],
        out_specs=[
            pl.BlockSpec((BR, D), lambda i: (i, 0)),
            pl.BlockSpec((BR, 1), lambda i: (i, 0)),
        ],
        out_shape=[
            jax.ShapeDtypeStruct((n_pad, D), F32),
            jax.ShapeDtypeStruct((n_pad, 1), F32),
        ],
    )(x_p, W, degs3, degs3)


def _tc_combine(s_all, xwn, dinv, b, n):
    D = xwn.shape[1]
    BR = 1000

    def body(s0_ref, s1_ref, xwn_ref, dinv_ref, b_ref, out_ref):
        acc = s0_ref[0] + s1_ref[0] + xwn_ref[...]
        out_ref[...] = jnp.maximum(acc * dinv_ref[...] + b_ref[...], 0.0)

    return pl.pallas_call(
        body,
        grid=(n // BR,),
        in_specs=[
            pl.BlockSpec((1, BR, D), lambda i: (0, i, 0)),
            pl.BlockSpec((1, BR, D), lambda i: (1, i, 0)),
            pl.BlockSpec((BR, D), lambda i: (i, 0)),
            pl.BlockSpec((BR, 1), lambda i: (i, 0)),
            pl.BlockSpec((D,), lambda i: (0,)),
        ],
        out_specs=pl.BlockSpec((BR, D), lambda i: (i, 0)),
        out_shape=jax.ShapeDtypeStruct((n, D), F32),
    )(s_all, s_all, xwn, dinv, b)


def kernel(x, edge_index, W, b):
    n, D = x.shape
    E = edge_index.shape[1]
    n_pad = ((n + 255) // 256) * 256
    chunks = ((-(-E // (NW * CB)) + 7) // 8) * 8
    e_pad = NW * CB * chunks

    pad = jnp.full((e_pad - E,), n, I32)
    src3 = jnp.concatenate([edge_index[0], pad]).reshape(NW, chunks, CB)
    dst3 = jnp.concatenate([edge_index[1], pad]).reshape(NW, chunks, CB)
    x_p = jnp.pad(x, ((0, n_pad - n), (0, 0)))

    degs = _sc_degree(dst3, n_pad, chunks)
    xwn, dinv = _tc_prescale(x_p, W, degs.reshape(NC, n_pad, 1))
    s_all = _sc_aggregate(xwn, src3, dst3, n_pad, chunks)
    return _tc_combine(s_all, xwn, dinv, b, n)
